# Initial kernel scaffold; baseline (speedup 1.0000x reference)
import jax
import jax.numpy as jnp
from jax.experimental import pallas as pl


def kernel(x, emb, W, b):
    raise NotImplementedError("write your pallas kernel here")


# speedup vs baseline: 2.7815x; 2.7815x over previous
import functools

import jax
import jax.numpy as jnp
from jax import lax
from jax.experimental import pallas as pl
from jax.experimental.pallas import tpu as pltpu
from jax.experimental.pallas import tpu_sc as plsc


def _fold_body(emb_ref, w_ref, b_ref, out_ref):
    t = lax.dot_general(
        emb_ref[...], w_ref[...],
        (((1,), (1,)), ((), ())),
        preferred_element_type=jnp.float32,
        precision=lax.Precision.HIGHEST,
    )
    out_ref[...] = t + b_ref[...]


def _fold_table(emb, W, b2d):
    V, _ = emb.shape
    C = W.shape[0]
    return pl.pallas_call(
        _fold_body,
        out_shape=jax.ShapeDtypeStruct((V, C), jnp.float32),
    )(emb, W, b2d)


def _gather_rows(table, idx_flat):
    V, C = table.shape
    B = idx_flat.shape[0]
    info = plsc.get_sparse_core_info()
    nc, ns = info.num_cores, info.num_subcores
    nw = nc * ns
    b_per_w = B // nw
    G = 128
    NG = 25
    CH = G * NG
    n_ch = b_per_w // CH
    mesh = plsc.VectorSubcoreMesh(core_axis_name="c", subcore_axis_name="s")

    @functools.partial(
        pl.kernel,
        mesh=mesh,
        out_type=jax.ShapeDtypeStruct((B, C), jnp.float32),
        scratch_types=[
            pltpu.VMEM((NG, G), jnp.int32),
            pltpu.VMEM((CH, C), jnp.float32),
            pltpu.SemaphoreType.DMA,
        ],
        compiler_params=pltpu.CompilerParams(use_tc_tiling_on_sc=False),
    )
    def k(table_hbm, idx_hbm, out_hbm, idx_v, rows_v, sem):
        wid = lax.axis_index("s") * nc + lax.axis_index("c")
        base = wid * b_per_w
        base_g = wid * (b_per_w // G)

        def body(i, carry):
            off = base + i * CH
            pltpu.sync_copy(idx_hbm.at[pl.ds(base_g + i * NG, NG)], idx_v)
            for j in range(NG):
                pltpu.async_copy(
                    table_hbm.at[idx_v.at[j]],
                    rows_v.at[pl.ds(j * G, G)],
                    sem,
                )
            for j in range(NG):
                pltpu.make_async_copy(
                    table_hbm.at[idx_v.at[j]],
                    rows_v.at[pl.ds(j * G, G)],
                    sem,
                ).wait()
            pltpu.sync_copy(rows_v, out_hbm.at[pl.ds(off, CH)])
            return carry

        lax.fori_loop(0, n_ch, body, 0)

    return k(table, idx_flat.reshape(B // G, G))


def _round_up(n, m):
    return (n + m - 1) // m * m


def kernel(x, emb, W, b):
    C = W.shape[0]
    Cp = _round_up(C, 32)
    Wp = jnp.pad(W.astype(jnp.float32), ((0, Cp - C), (0, 0)))
    bp = jnp.pad(b.astype(jnp.float32), (0, Cp - C)).reshape(1, Cp)
    tab = _fold_table(emb, Wp, bp)
    idx = x.reshape(-1).astype(jnp.int32)
    out = _gather_rows(tab, idx)
    return out[:, :C].reshape(x.shape[0], x.shape[1], C)
